# baseline (speedup 1.0000x reference)
import functools

import jax
import jax.numpy as jnp
import numpy as np
from jax.experimental import pallas as pl
from jax.experimental.pallas import tpu as pltpu

_TINY = np.float32(np.finfo(np.float32).tiny)
_KS0 = np.uint32(0)
_KS1 = np.uint32(42)
_KS2 = np.uint32(0x1BD11BDA) ^ _KS1
_NEG_INF = np.float32(-np.inf)
_I32_MAX = np.int32(0x7FFFFFFF)


def _rotl(x, d):
    return (x << np.uint32(d)) | (x >> np.uint32(32 - d))


def _threefry_bits(x1_init):
    x1 = x1_init
    x0 = x1
    t = _rotl(x1, 13)
    x1 = x0 ^ t

    def rounds(x0, x1, rots):
        for r in rots:
            x0 = x0 + x1
            x1 = _rotl(x1, r)
            x1 = x0 ^ x1
        return x0, x1

    x0, x1 = rounds(x0, x1, (15, 26, 6))
    x0, x1 = x0 + _KS1, x1 + (_KS2 + np.uint32(1))
    x0, x1 = rounds(x0, x1, (17, 29, 16, 24))
    x0, x1 = x0 + _KS2, x1 + (_KS0 + np.uint32(2))
    x0, x1 = rounds(x0, x1, (13, 15, 26, 6))
    x0, x1 = x0 + _KS0, x1 + (_KS1 + np.uint32(3))
    x0, x1 = rounds(x0, x1, (17, 29, 16, 24))
    x0, x1 = x0 + _KS1, x1 + (_KS2 + np.uint32(4))
    x0, x1 = rounds(x0, x1, (13, 15, 26, 6))
    x0, x1 = x0 + _KS2, x1 + (_KS0 + np.uint32(5))
    return x0 ^ x1


def _gumbel_plus(x, x1_init):
    bits = _threefry_bits(x1_init)
    float_bits = (bits >> np.uint32(9)) | np.uint32(0x3F800000)
    u0 = jax.lax.bitcast_convert_type(float_bits, jnp.float32) - np.float32(1.0)
    u = jnp.maximum(_TINY, u0 * (np.float32(1.0) - _TINY) + _TINY)
    g = -jnp.log(-jnp.log(u))
    return g + x


def _group_kernel(x_ref, o_ref, act_ref, *, rows_per_grp, sub, chunk, vocab,
                  lanes, grps):
    g = pl.program_id(0)
    t = rows_per_grp * sub
    nfull = chunk // lanes
    tail = chunk - nfull * lanes
    off_t = nfull * lanes

    qi = jax.lax.broadcasted_iota(jnp.uint32, (t, lanes), 0)
    li = jax.lax.broadcasted_iota(jnp.uint32, (t, lanes), 1)
    q = qi // np.uint32(sub)
    s = qi % np.uint32(sub)
    pat = q * np.uint32(vocab) + s * np.uint32(chunk) + li + _KS1
    vpat = (s * np.uint32(chunk) + li).astype(jnp.int32)

    qcol = jax.lax.broadcasted_iota(jnp.int32, (t, 1), 0) // np.int32(sub)
    a32 = jnp.full((t, 1), act_ref[0], jnp.int32)
    for r in range(1, rows_per_grp):
        a32 = jnp.where(qcol == r, act_ref[r], a32)
    one = np.float32(1.0)
    zero = np.float32(0.0)
    for c in range(nfull):
        off = c * lanes
        o_ref[0, :, off:off + lanes] = jnp.where(
            vpat == a32 - np.int32(off), one, zero)
    o_ref[0, :, off_t:chunk] = jnp.where(
        vpat[:, :tail] == a32 - np.int32(off_t), one, zero)

    base = jnp.minimum(g, grps - 1).astype(jnp.uint32) \
        * np.uint32(rows_per_grp * vocab)
    run_max = jnp.full((t, lanes), _NEG_INF, jnp.float32)
    run_idx = jnp.full((t, lanes), _I32_MAX, jnp.int32)
    for c in range(nfull):
        x = x_ref[0, :, c * lanes:(c + 1) * lanes]
        x1v = pat + (base + np.uint32(c * lanes))
        y = _gumbel_plus(x, x1v)
        upd = y > run_max
        run_idx = jnp.where(upd, x1v.astype(jnp.int32), run_idx)
        run_max = jnp.maximum(run_max, y)
    x1t = pat[:, :tail] + (base + np.uint32(off_t))
    y_t = _gumbel_plus(x_ref[0, :, off_t:chunk], x1t)
    pad = lanes - tail
    y_full = jnp.concatenate(
        [y_t, jnp.full((t, pad), _NEG_INF, jnp.float32)], axis=1)
    v_full = jnp.concatenate(
        [x1t.astype(jnp.int32), jnp.full((t, pad), _I32_MAX, jnp.int32)],
        axis=1)
    upd_t = y_full > run_max
    run_idx = jnp.where(upd_t, v_full, run_idx)
    run_max = jnp.maximum(run_max, y_full)

    for r in range(rows_per_grp):
        rm = run_max[r * sub:(r + 1) * sub]
        ri = run_idx[r * sub:(r + 1) * sub]
        m = jnp.max(rm)
        w = jnp.min(jnp.where(rm == m, ri, _I32_MAX))
        act_ref[r] = w - (base + np.uint32(r * vocab) + _KS1
                       ).astype(jnp.int32)


def kernel(x):
    m, n, vocab = x.shape
    rows = m * n
    sub = 8
    rows_per_grp = 4
    grps = rows // rows_per_grp
    chunk = vocab // sub
    t = rows_per_grp * sub
    xr = x.reshape(grps, t, chunk)
    out = pl.pallas_call(
        functools.partial(_group_kernel, rows_per_grp=rows_per_grp, sub=sub,
                          chunk=chunk, vocab=vocab, lanes=512, grps=grps),
        grid=(grps + 1,),
        in_specs=[pl.BlockSpec(
            (1, t, chunk), lambda g: (jnp.minimum(g, grps - 1), 0, 0))],
        out_specs=pl.BlockSpec(
            (1, t, chunk), lambda g: (jnp.maximum(g - 1, 0), 0, 0)),
        out_shape=jax.ShapeDtypeStruct((grps, t, chunk), jnp.float32),
        scratch_shapes=[pltpu.SMEM((rows_per_grp,), jnp.int32)],
    )(xr)
    return out.reshape(m, n, vocab)


# speedup vs baseline: 1.0095x; 1.0095x over previous
import functools

import jax
import jax.numpy as jnp
import numpy as np
from jax.experimental import pallas as pl
from jax.experimental.pallas import tpu as pltpu

_TINY = np.float32(np.finfo(np.float32).tiny)
_KS0 = np.uint32(0)
_KS1 = np.uint32(42)
_KS2 = np.uint32(0x1BD11BDA) ^ _KS1
_NEG_INF = np.float32(-np.inf)
_I32_MAX = np.int32(0x7FFFFFFF)


def _rotl(x, d):
    return (x << np.uint32(d)) | (x >> np.uint32(32 - d))


def _threefry_bits(x1_init):
    x1 = x1_init
    x0 = x1
    t = _rotl(x1, 13)
    x1 = x0 ^ t

    def rounds(x0, x1, rots):
        for r in rots:
            x0 = x0 + x1
            x1 = _rotl(x1, r)
            x1 = x0 ^ x1
        return x0, x1

    x0, x1 = rounds(x0, x1, (15, 26, 6))
    x0, x1 = x0 + _KS1, x1 + (_KS2 + np.uint32(1))
    x0, x1 = rounds(x0, x1, (17, 29, 16, 24))
    x0, x1 = x0 + _KS2, x1 + (_KS0 + np.uint32(2))
    x0, x1 = rounds(x0, x1, (13, 15, 26, 6))
    x0, x1 = x0 + _KS0, x1 + (_KS1 + np.uint32(3))
    x0, x1 = rounds(x0, x1, (17, 29, 16, 24))
    x0, x1 = x0 + _KS1, x1 + (_KS2 + np.uint32(4))
    x0, x1 = rounds(x0, x1, (13, 15, 26, 6))
    x0, x1 = x0 + _KS2, x1 + (_KS0 + np.uint32(5))
    return x0 ^ x1


def _gumbel_plus(x, x1_init):
    bits = _threefry_bits(x1_init)
    float_bits = (bits >> np.uint32(9)) | np.uint32(0x3F800000)
    u0 = jax.lax.bitcast_convert_type(float_bits, jnp.float32) - np.float32(1.0)
    u = jnp.maximum(_TINY, u0 * (np.float32(1.0) - _TINY) + _TINY)
    g = -jnp.log(-jnp.log(u))
    return g + x


def _group_kernel(x_ref, o_ref, *, rows_per_grp, sub, chunk, vocab, lanes):
    g = pl.program_id(0)
    t = rows_per_grp * sub
    nfull = chunk // lanes
    tail = chunk - nfull * lanes
    off_t = nfull * lanes

    qi = jax.lax.broadcasted_iota(jnp.uint32, (t, lanes), 0)
    li = jax.lax.broadcasted_iota(jnp.uint32, (t, lanes), 1)
    q = qi // np.uint32(sub)
    s = qi % np.uint32(sub)
    pat = q * np.uint32(vocab) + s * np.uint32(chunk) + li + _KS1
    vpat = (s * np.uint32(chunk) + li).astype(jnp.int32)

    base = g.astype(jnp.uint32) * np.uint32(rows_per_grp * vocab)
    run_max = jnp.full((t, lanes), _NEG_INF, jnp.float32)
    run_idx = jnp.full((t, lanes), _I32_MAX, jnp.int32)
    for c in range(nfull):
        x = x_ref[0, :, c * lanes:(c + 1) * lanes]
        x1v = pat + (base + np.uint32(c * lanes))
        y = _gumbel_plus(x, x1v)
        upd = y > run_max
        run_idx = jnp.where(upd, x1v.astype(jnp.int32), run_idx)
        run_max = jnp.maximum(run_max, y)
    x1t = pat[:, :tail] + (base + np.uint32(off_t))
    y_t = _gumbel_plus(x_ref[0, :, off_t:chunk], x1t)
    pad = lanes - tail
    y_full = jnp.concatenate(
        [y_t, jnp.full((t, pad), _NEG_INF, jnp.float32)], axis=1)
    v_full = jnp.concatenate(
        [x1t.astype(jnp.int32), jnp.full((t, pad), _I32_MAX, jnp.int32)],
        axis=1)
    upd_t = y_full > run_max
    run_idx = jnp.where(upd_t, v_full, run_idx)
    run_max = jnp.maximum(run_max, y_full)

    acts = []
    for r in range(rows_per_grp):
        rm = run_max[r * sub:(r + 1) * sub]
        ri = run_idx[r * sub:(r + 1) * sub]
        m = jnp.max(rm)
        w = jnp.min(jnp.where(rm == m, ri, _I32_MAX))
        acts.append(w - (base + np.uint32(r * vocab) + _KS1
                         ).astype(jnp.int32))

    qcol = jax.lax.broadcasted_iota(jnp.int32, (t, 1), 0) // np.int32(sub)
    a32 = jnp.full((t, 1), acts[0], jnp.int32)
    for r in range(1, rows_per_grp):
        a32 = jnp.where(qcol == r, acts[r], a32)
    one = np.float32(1.0)
    zero = np.float32(0.0)
    for c in range(nfull):
        off = c * lanes
        o_ref[0, :, off:off + lanes] = jnp.where(
            vpat == a32 - np.int32(off), one, zero)
    o_ref[0, :, off_t:chunk] = jnp.where(
        vpat[:, :tail] == a32 - np.int32(off_t), one, zero)


def kernel(x):
    m, n, vocab = x.shape
    rows = m * n
    sub = 8
    rows_per_grp = 4
    grps = rows // rows_per_grp
    chunk = vocab // sub
    t = rows_per_grp * sub
    xr = x.reshape(grps, t, chunk)
    out = pl.pallas_call(
        functools.partial(_group_kernel, rows_per_grp=rows_per_grp, sub=sub,
                          chunk=chunk, vocab=vocab, lanes=512),
        grid=(grps,),
        in_specs=[pl.BlockSpec((1, t, chunk), lambda g: (g, 0, 0))],
        out_specs=pl.BlockSpec((1, t, chunk), lambda g: (g, 0, 0)),
        out_shape=jax.ShapeDtypeStruct((grps, t, chunk), jnp.float32),
        compiler_params=pltpu.CompilerParams(
            dimension_semantics=("arbitrary",)),
    )(xr)
    return out.reshape(m, n, vocab)


# speedup vs baseline: 1.0184x; 1.0088x over previous
import functools

import jax
import jax.numpy as jnp
import numpy as np
from jax.experimental import pallas as pl
from jax.experimental.pallas import tpu as pltpu

_TINY = np.float32(np.finfo(np.float32).tiny)
_KS0 = np.uint32(0)
_KS1 = np.uint32(42)
_KS2 = np.uint32(0x1BD11BDA) ^ _KS1
_NEG_INF = np.float32(-np.inf)
_I32_MAX = np.int32(0x7FFFFFFF)


def _rotl(x, d):
    return (x << np.uint32(d)) | (x >> np.uint32(32 - d))


def _threefry_bits(x1_init):
    x1 = x1_init
    x0 = x1
    t = _rotl(x1, 13)
    x1 = x0 ^ t

    def rounds(x0, x1, rots):
        for r in rots:
            x0 = x0 + x1
            x1 = _rotl(x1, r)
            x1 = x0 ^ x1
        return x0, x1

    x0, x1 = rounds(x0, x1, (15, 26, 6))
    x0, x1 = x0 + _KS1, x1 + (_KS2 + np.uint32(1))
    x0, x1 = rounds(x0, x1, (17, 29, 16, 24))
    x0, x1 = x0 + _KS2, x1 + (_KS0 + np.uint32(2))
    x0, x1 = rounds(x0, x1, (13, 15, 26, 6))
    x0, x1 = x0 + _KS0, x1 + (_KS1 + np.uint32(3))
    x0, x1 = rounds(x0, x1, (17, 29, 16, 24))
    x0, x1 = x0 + _KS1, x1 + (_KS2 + np.uint32(4))
    x0, x1 = rounds(x0, x1, (13, 15, 26, 6))
    x0, x1 = x0 + _KS2, x1 + (_KS0 + np.uint32(5))
    return x0 ^ x1


def _gumbel_plus(x, x1_init):
    bits = _threefry_bits(x1_init)
    float_bits = (bits >> np.uint32(9)) | np.uint32(0x3F800000)
    u0 = jax.lax.bitcast_convert_type(float_bits, jnp.float32) - np.float32(1.0)
    u = jnp.maximum(_TINY, u0 * (np.float32(1.0) - _TINY) + _TINY)
    g = -jnp.log(-jnp.log(u))
    return g + x


def _group_kernel(x_ref, o_ref, *, rows_per_grp, sub, chunk, vocab, lanes):
    g = pl.program_id(0)
    t = rows_per_grp * sub
    nfull = chunk // lanes
    tail = chunk - nfull * lanes
    off_t = nfull * lanes

    qi = jax.lax.broadcasted_iota(jnp.uint32, (t, lanes), 0)
    li = jax.lax.broadcasted_iota(jnp.uint32, (t, lanes), 1)
    q = qi // np.uint32(sub)
    s = qi % np.uint32(sub)
    pat = q * np.uint32(vocab) + s * np.uint32(chunk) + li + _KS1
    vpat = (s * np.uint32(chunk) + li).astype(jnp.int32)

    base = g.astype(jnp.uint32) * np.uint32(rows_per_grp * vocab)
    run_max = jnp.full((t, lanes), _NEG_INF, jnp.float32)
    run_idx = jnp.full((t, lanes), _I32_MAX, jnp.int32)
    for c in range(nfull):
        x = x_ref[0, :, c * lanes:(c + 1) * lanes]
        x1v = pat + (base + np.uint32(c * lanes))
        y = _gumbel_plus(x, x1v)
        upd = y > run_max
        run_idx = jnp.where(upd, x1v.astype(jnp.int32), run_idx)
        run_max = jnp.maximum(run_max, y)
    x1t = pat[:, :tail] + (base + np.uint32(off_t))
    y_t = _gumbel_plus(x_ref[0, :, off_t:chunk], x1t)
    pad = lanes - tail
    y_full = jnp.concatenate(
        [y_t, jnp.full((t, pad), _NEG_INF, jnp.float32)], axis=1)
    v_full = jnp.concatenate(
        [x1t.astype(jnp.int32), jnp.full((t, pad), _I32_MAX, jnp.int32)],
        axis=1)
    upd_t = y_full > run_max
    run_idx = jnp.where(upd_t, v_full, run_idx)
    run_max = jnp.maximum(run_max, y_full)

    acts = []
    for r in range(rows_per_grp):
        rm = run_max[r * sub:(r + 1) * sub]
        ri = run_idx[r * sub:(r + 1) * sub]
        m = jnp.max(rm)
        w = jnp.min(jnp.where(rm == m, ri, _I32_MAX))
        acts.append(w - (base + np.uint32(r * vocab) + _KS1
                         ).astype(jnp.int32))

    qcol = jax.lax.broadcasted_iota(jnp.int32, (t, 1), 0) // np.int32(sub)
    a32 = jnp.full((t, 1), acts[0], jnp.int32)
    for r in range(1, rows_per_grp):
        a32 = jnp.where(qcol == r, acts[r], a32)
    one = np.float32(1.0)
    zero = np.float32(0.0)
    for c in range(nfull):
        off = c * lanes
        o_ref[0, :, off:off + lanes] = jnp.where(
            vpat == a32 - np.int32(off), one, zero)
    o_ref[0, :, off_t:chunk] = jnp.where(
        vpat[:, :tail] == a32 - np.int32(off_t), one, zero)


def kernel(x):
    m, n, vocab = x.shape
    rows = m * n
    sub = 8
    rows_per_grp = 4
    grps = rows // rows_per_grp
    chunk = vocab // sub
    t = rows_per_grp * sub
    xr = x.reshape(grps, t, chunk)
    out = pl.pallas_call(
        functools.partial(_group_kernel, rows_per_grp=rows_per_grp, sub=sub,
                          chunk=chunk, vocab=vocab, lanes=256),
        grid=(grps,),
        in_specs=[pl.BlockSpec((1, t, chunk), lambda g: (g, 0, 0))],
        out_specs=pl.BlockSpec((1, t, chunk), lambda g: (g, 0, 0)),
        out_shape=jax.ShapeDtypeStruct((grps, t, chunk), jnp.float32),
        compiler_params=pltpu.CompilerParams(
            dimension_semantics=("arbitrary",)),
    )(xr)
    return out.reshape(m, n, vocab)
